# baseline (speedup 1.0000x reference)
import numpy as np
import jax
import jax.numpy as jnp
from jax.experimental import pallas as pl
from jax.experimental.pallas import tpu as pltpu

_IN1 = 64
_IN2 = 64
_OUT = 64
_SAMPLE = 2
_LOG2PI = float(np.log(2.0 * np.pi))


def _phase1_body(aap_ref, aue_ref, h_ref, gap_ref, gfaap_ref, gue_ref, gfaue_ref,
                 q1ap_ref, q2ap_ref, q1ue_ref, q2ue_ref, p1ap_ref, p1ue_ref,
                 attap_ref, attue_ref, uap_ref, uue_ref,
                 preap_ref, preue_ref, alap_ref, alue_ref):
    f32 = jnp.float32
    n1 = aap_ref.shape[2]
    n2 = aue_ref.shape[2]

    def side(a, idx_i, idx_j, gfa, att, u, n_nodes, deg, n_edges):
        iota_n = jax.lax.broadcasted_iota(jnp.int32, (n_nodes, n_edges), 0)
        oh_i = (iota_n == idx_i).astype(f32)
        oh_j = (iota_n == idx_j).astype(f32)
        xiT = jnp.dot(a, oh_i, preferred_element_type=f32)
        xjT = jnp.dot(a, oh_j, preferred_element_type=f32)
        s = (jnp.dot(att[:, :_IN1], xiT, preferred_element_type=f32)
             + jnp.dot(att[:, _IN1:], xjT, preferred_element_type=f32))
        lrelu = jnp.where(s >= 0, s, 0.2 * s)
        alpha = jnp.clip(jax.nn.sigmoid(lrelu), 0.01, 0.99)
        logits = jnp.log(alpha) - jnp.log1p(-alpha)
        noise = jnp.log(u) - jnp.log1p(-u)
        bern = jax.nn.sigmoid((logits + noise) / 0.1)
        xg = xjT * bern
        iota_e = jax.lax.broadcasted_iota(jnp.int16, (n_nodes, n_edges), 1)
        gfa16 = gfa.astype(jnp.int16)
        cmat = jnp.zeros((n_nodes, n_edges), jnp.bfloat16)
        for k in range(deg):
            cmat = cmat + (gfa16[:, k:k + 1] == iota_e).astype(jnp.bfloat16)
        agg = jax.lax.dot_general(xg, cmat.astype(f32),
                                  (((1,), (1,)), ((), ())),
                                  preferred_element_type=f32)
        return agg, alpha

    a_ap = aap_ref[0]
    a_ue = aue_ref[0]
    e_ap = gap_ref.shape[2]
    e_ue = gue_ref.shape[2]
    agg_ap, alpha_ap = side(a_ap, gap_ref[0, 1:2, :], gap_ref[0, 0:1, :],
                            gfaap_ref[0], attap_ref[...], uap_ref[0],
                            n1, gfaap_ref.shape[2], e_ap)
    agg_ue, alpha_ue = side(a_ue, gue_ref[0, 1:2, :], gue_ref[0, 0:1, :],
                            gfaue_ref[0], attue_ref[...], uue_ref[0],
                            n2, gfaue_ref.shape[2], e_ue)

    h = h_ref[0]
    hm1 = jnp.sum(h, axis=2) * (1.0 / n2)
    hm2 = jnp.sum(h, axis=1) * (1.0 / n1)

    mean_ap = jnp.sum(agg_ap, axis=1, keepdims=True) * (1.0 / n1)
    mean_ue = jnp.sum(agg_ue, axis=1, keepdims=True) * (1.0 / n2)

    f = jnp.float32
    a1 = jnp.dot(q1ap_ref[...], agg_ap, preferred_element_type=f)
    a2 = jnp.dot(q2ap_ref[...], mean_ue, preferred_element_type=f)
    a3 = jnp.dot(p1ap_ref[...], hm1, preferred_element_type=f)
    preap_ref[0] = jnp.maximum(2.0 * a1 + 2.0 * a2 + 0.1 * a3, 0.0)

    u1 = jnp.dot(q1ue_ref[...], agg_ue, preferred_element_type=f)
    u2 = jnp.dot(q2ue_ref[...], mean_ap, preferred_element_type=f)
    u3 = jnp.dot(p1ue_ref[...], hm2, preferred_element_type=f)
    preue_ref[0] = jnp.maximum(2.0 * u1 + 2.0 * u2 + 0.1 * u3, 0.0)

    alap_ref[0] = alpha_ap
    alue_ref[0] = alpha_ue


def _phase2_body(preap_ref, preue_ref, alap_ref, alue_ref,
                 epsap_ref, epsue_ref, g_ref, b_ref,
                 oap_ref, oue_ref, ixzap_ref, ixzue_ref, iazap_ref, iazue_ref):
    gamma = g_ref[:, 0:1]
    beta = b_ref[:, 0:1]

    def bn(x):
        bsz = x.shape[0] * x.shape[2]
        s = jnp.sum(jnp.sum(x, axis=2, keepdims=True), axis=0, keepdims=True)
        m = s * (1.0 / bsz)
        d = x - m
        v = jnp.sum(jnp.sum(d * d, axis=2, keepdims=True), axis=0,
                    keepdims=True) * (1.0 / bsz)
        return gamma[None] * d / jnp.sqrt(v + 1e-5) + beta[None]

    def ib(y, eps_ref):
        mean = y[:, :_OUT, :]
        std = jax.nn.softplus(y[:, _OUT:, :]) + 1e-10
        logstd = jnp.log(std)
        acc = None
        for si in range(_SAMPLE):
            z = mean + std * eps_ref[si]
            e1 = -((z - mean) ** 2) / (2.0 * std * std) - logstd
            diff = jnp.sum(e1 + 0.5 * z * z, axis=1)
            acc = diff if acc is None else acc + diff
        t = acc * (1.0 / _SAMPLE)
        return jnp.sum(t, axis=1, keepdims=True)

    def kl(al_ref):
        al = al_ref[:, 0, :]
        term = (al * jnp.log(al / 0.5)
                + (1.0 - al) * jnp.log((1.0 - al) / 0.5))
        return jnp.sum(term, axis=1, keepdims=True)

    bsz = preap_ref.shape[0]
    w = ixzap_ref.shape[1]
    y_ap = bn(preap_ref[...])
    y_ue = bn(preue_ref[...])
    oap_ref[...] = y_ap
    oue_ref[...] = y_ue
    ixzap_ref[...] = jnp.broadcast_to(ib(y_ap, epsap_ref), (bsz, w))
    ixzue_ref[...] = jnp.broadcast_to(ib(y_ue, epsue_ref), (bsz, w))
    iazap_ref[...] = jnp.broadcast_to(kl(alap_ref), (bsz, w))
    iazue_ref[...] = jnp.broadcast_to(kl(alue_ref), (bsz, w))


def kernel(A_AP, A_UE, H, Graph_AP_reshape, GFA_AP, Graph_UE_reshape, GFA_UE,
           Q1_AP, Q2_AP, Q1_UE, Q2_UE, P1_AP, P1_UE, Att_AP, Att_UE,
           bn_gamma, bn_beta, permutation_size1, permutation_size2, BATCH_SIZE):
    f32 = jnp.float32
    B, D, N1 = A_AP.shape
    N2 = A_UE.shape[2]
    E_ap = Graph_AP_reshape.shape[2]
    E_ue = Graph_UE_reshape.shape[2]
    deg_ap = GFA_AP.shape[2]
    deg_ue = GFA_UE.shape[2]
    O2 = Q1_AP.shape[0]

    kr = jax.random.key(42)
    k1, k2, k3, k4 = jax.random.split(kr, 4)
    u_ap = jax.random.uniform(k1, (B, E_ap), minval=1e-6, maxval=1.0 - 1e-6)
    u_ue = jax.random.uniform(k2, (B, E_ue), minval=1e-6, maxval=1.0 - 1e-6)
    eps_ap = jax.random.normal(k3, (_SAMPLE, B * N1, _OUT))
    eps_ue = jax.random.normal(k4, (_SAMPLE, B * N2, _OUT))
    eps_ap_t = eps_ap.reshape(_SAMPLE, B, N1, _OUT).transpose(0, 1, 3, 2)
    eps_ue_t = eps_ue.reshape(_SAMPLE, B, N2, _OUT).transpose(0, 1, 3, 2)

    gap = Graph_AP_reshape.astype(jnp.int32)
    gue = Graph_UE_reshape.astype(jnp.int32)
    gfaap = GFA_AP.astype(jnp.int32)
    gfaue = GFA_UE.astype(jnp.int32)
    att_ap = Att_AP.reshape(1, 2 * D).astype(f32)
    att_ue = Att_UE.reshape(1, 2 * D).astype(f32)
    u_ap3 = u_ap.reshape(B, 1, E_ap)
    u_ue3 = u_ue.reshape(B, 1, E_ue)
    gcol = jnp.broadcast_to(bn_gamma.reshape(O2, 1), (O2, 128)).astype(f32)
    bcol = jnp.broadcast_to(bn_beta.reshape(O2, 1), (O2, 128)).astype(f32)

    wspec = lambda shp: pl.BlockSpec(shp, lambda b: (0,) * len(shp))
    bspec = lambda shp: pl.BlockSpec(shp, lambda b: (b,) + (0,) * (len(shp) - 1))

    pre_ap, pre_ue, alpha_ap, alpha_ue = pl.pallas_call(
        _phase1_body,
        grid=(B,),
        in_specs=[
            bspec((1, D, N1)),
            bspec((1, D, N2)),
            bspec((1, _IN2, N1, N2)),
            bspec((1, 2, E_ap)),
            bspec((1, N1, deg_ap)),
            bspec((1, 2, E_ue)),
            bspec((1, N2, deg_ue)),
            wspec((O2, D)),
            wspec((O2, D)),
            wspec((O2, D)),
            wspec((O2, D)),
            wspec((O2, _IN2)),
            wspec((O2, _IN2)),
            wspec((1, 2 * D)),
            wspec((1, 2 * D)),
            bspec((1, 1, E_ap)),
            bspec((1, 1, E_ue)),
        ],
        out_specs=[
            bspec((1, O2, N1)),
            bspec((1, O2, N2)),
            bspec((1, 1, E_ap)),
            bspec((1, 1, E_ue)),
        ],
        out_shape=[
            jax.ShapeDtypeStruct((B, O2, N1), f32),
            jax.ShapeDtypeStruct((B, O2, N2), f32),
            jax.ShapeDtypeStruct((B, 1, E_ap), f32),
            jax.ShapeDtypeStruct((B, 1, E_ue), f32),
        ],
        compiler_params=pltpu.CompilerParams(
            dimension_semantics=("arbitrary",)),
    )(A_AP, A_UE, H, gap, gfaap, gue, gfaue,
      Q1_AP, Q2_AP, Q1_UE, Q2_UE, P1_AP, P1_UE, att_ap, att_ue, u_ap3, u_ue3)

    out_ap, out_ue, ixz_ap, ixz_ue, iaz_ap, iaz_ue = pl.pallas_call(
        _phase2_body,
        out_shape=[
            jax.ShapeDtypeStruct((B, O2, N1), f32),
            jax.ShapeDtypeStruct((B, O2, N2), f32),
            jax.ShapeDtypeStruct((B, 128), f32),
            jax.ShapeDtypeStruct((B, 128), f32),
            jax.ShapeDtypeStruct((B, 128), f32),
            jax.ShapeDtypeStruct((B, 128), f32),
        ],
    )(pre_ap, pre_ue, alpha_ap, alpha_ue, eps_ap_t, eps_ue_t, gcol, bcol)

    return (out_ap, out_ue, ixz_ap[:, 0], ixz_ue[:, 0],
            iaz_ap[:, 0], iaz_ue[:, 0])


# speedup vs baseline: 1.2027x; 1.2027x over previous
import numpy as np
import jax
import jax.numpy as jnp
from jax import lax
from jax.experimental import pallas as pl
from jax.experimental.pallas import tpu as pltpu
from jax.experimental.pallas import tpu_sc as plsc

_IN1 = 64
_IN2 = 64
_OUT = 64
_SAMPLE = 2
_LOGIT_CLIP = float(np.log(0.99) - np.log(0.01))
_NC = 2
_NS = 16
_NW = _NC * _NS


def _pre_body(aap_ref, aue_ref, attap_ref, attue_ref, uap_ref, uue_ref,
              pap_ref, pue_ref, nap_ref, nue_ref):
    f32 = jnp.float32
    bsz = aap_ref.shape[0]
    for b in range(bsz):
        for (a_ref, att_ref, p_ref) in ((aap_ref, attap_ref, pap_ref),
                                        (aue_ref, attue_ref, pue_ref)):
            a = a_ref[b]
            p_ref[b, 0:1, :] = jnp.dot(att_ref[:, :_IN1], a,
                                       preferred_element_type=f32)
            p_ref[b, 1:2, :] = jnp.dot(att_ref[:, _IN1:], a,
                                       preferred_element_type=f32)
    for (u_ref, n_ref) in ((uap_ref, nap_ref), (uue_ref, nue_ref)):
        u = u_ref[...]
        n_ref[...] = jnp.log(u) - jnp.log1p(-u)


def _make_sc_edge(bsz, n, deg, d, n_edges):
    tiles_per_b = _NW // bsz
    npt = n // tiles_per_b
    ept = n_edges // tiles_per_b
    mesh = plsc.VectorSubcoreMesh(core_axis_name="c", subcore_axis_name="s")
    f32 = jnp.float32
    i32 = jnp.int32

    npad = max(n, 128)

    def body(at_hbm, g_hbm, p_hbm, no_hbm, gfa_hbm, agg_hbm, al_hbm,
             at_v, gi_v, gj_v, pi_v, pj_v, no_v, gfa_v, bern_v, al_v, agg_v):
        wid = lax.axis_index("s") * _NC + lax.axis_index("c")
        b = wid // tiles_per_b
        t = wid % tiles_per_b
        pltpu.sync_copy(at_hbm.at[b], at_v)
        pltpu.sync_copy(g_hbm.at[b, 0], gj_v)
        pltpu.sync_copy(g_hbm.at[b, 1], gi_v)
        pltpu.sync_copy(p_hbm.at[b, 0], pi_v)
        pltpu.sync_copy(p_hbm.at[b, 1], pj_v)
        pltpu.sync_copy(no_hbm.at[b], no_v)
        pltpu.sync_copy(gfa_hbm.at[b], gfa_v)

        def phase_a(g, carry):
            base = g * 16
            ii = gi_v[pl.ds(base, 16)]
            jj = gj_v[pl.ds(base, 16)]
            s_e = plsc.load_gather(pi_v, [ii]) + plsc.load_gather(pj_v, [jj])
            lr = jnp.where(s_e >= 0.0, s_e, 0.2 * s_e)
            alpha = jnp.clip(1.0 / (1.0 + jnp.exp(-lr)), 0.01, 0.99)
            logits = jnp.clip(lr, -_LOGIT_CLIP, _LOGIT_CLIP)
            y = (logits + no_v[pl.ds(base, 16)]) * 10.0
            bern_v[pl.ds(base, 16)] = 1.0 / (1.0 + jnp.exp(-y))
            al_v[pl.ds(base, 16)] = alpha
            return carry

        lax.fori_loop(0, n_edges // 16, phase_a, 0)

        for ng in range(npt // 16):
            n0 = t * npt + ng * 16
            nl_v = lax.iota(i32, 16) + ng * 16
            for half in range(d // 32):
                dof = half * 32

                def phase_b(k, accs):
                    e_v = gfa_v[pl.ds(k * n + n0, 16)]
                    j_v = plsc.load_gather(gj_v, [e_v])
                    b_v = plsc.load_gather(bern_v, [e_v])
                    out = []
                    for dd in range(32):
                        dv = jnp.full((16,), dd + dof, i32)
                        x = plsc.load_gather(at_v, [j_v, dv])
                        out.append(accs[dd] + x * b_v)
                    return tuple(out)

                accs0 = tuple(jnp.zeros((16,), f32) for _ in range(32))
                accs = lax.fori_loop(0, deg, phase_b, accs0)
                for dd in range(32):
                    dv = jnp.full((16,), dd + dof, i32)
                    plsc.store_scatter(agg_v, [nl_v, dv], accs[dd])

        pltpu.sync_copy(agg_v, agg_hbm.at[b, pl.ds(t * npt, npt)])
        pltpu.sync_copy(al_v.at[pl.ds(t * ept, ept)],
                        al_hbm.at[b, pl.ds(t * ept, ept)])

    return pl.kernel(
        body,
        out_type=(jax.ShapeDtypeStruct((bsz, n, d), f32),
                  jax.ShapeDtypeStruct((bsz, n_edges), f32)),
        mesh=mesh,
        compiler_params=pltpu.CompilerParams(needs_layout_passes=False),
        scratch_types=(
            pltpu.VMEM((n, d), f32),
            pltpu.VMEM((n_edges,), i32),
            pltpu.VMEM((n_edges,), i32),
            pltpu.VMEM((npad,), f32),
            pltpu.VMEM((npad,), f32),
            pltpu.VMEM((n_edges,), f32),
            pltpu.VMEM((deg * n,), i32),
            pltpu.VMEM((n_edges,), f32),
            pltpu.VMEM((n_edges,), f32),
            pltpu.VMEM((npt, d), f32),
        ),
    )


def _hmean_body(h_ref, hm1_ref, hm2_ref):
    h = h_ref[0]
    n1 = h.shape[1]
    n2 = h.shape[2]
    hm1_ref[0] = jnp.sum(h, axis=2) * (1.0 / n2)
    hm2_ref[0] = jnp.sum(h, axis=1) * (1.0 / n1)


def _final_body(aggap_ref, aggue_ref, hm1_ref, hm2_ref, alap_ref, alue_ref,
                epsap_ref, epsue_ref,
                q1ap_ref, q2ap_ref, q1ue_ref, q2ue_ref, p1ap_ref, p1ue_ref,
                g_ref, b_ref,
                oap_ref, oue_ref, ixzap_ref, ixzue_ref, iazap_ref, iazue_ref):
    f32 = jnp.float32
    bsz = aggap_ref.shape[0]
    n1 = aggap_ref.shape[1]
    n2 = aggue_ref.shape[1]
    gamma = g_ref[:, 0:1]
    beta = b_ref[:, 0:1]

    for b in range(bsz):
        agg_ap = aggap_ref[b]
        agg_ue = aggue_ref[b]
        m_ap = jnp.sum(agg_ap, axis=0, keepdims=True) * (1.0 / n1)
        m_ue = jnp.sum(agg_ue, axis=0, keepdims=True) * (1.0 / n2)
        dn = (((1,), (1,)), ((), ()))
        a1 = lax.dot_general(q1ap_ref[...], agg_ap, dn,
                             preferred_element_type=f32)
        a2 = lax.dot_general(q2ap_ref[...], m_ue, dn,
                             preferred_element_type=f32)
        a3 = jnp.dot(p1ap_ref[...], hm1_ref[b], preferred_element_type=f32)
        oap_ref[b] = jnp.maximum(2.0 * a1 + 2.0 * a2 + 0.1 * a3, 0.0)
        u1 = lax.dot_general(q1ue_ref[...], agg_ue, dn,
                             preferred_element_type=f32)
        u2 = lax.dot_general(q2ue_ref[...], m_ap, dn,
                             preferred_element_type=f32)
        u3 = jnp.dot(p1ue_ref[...], hm2_ref[b], preferred_element_type=f32)
        oue_ref[b] = jnp.maximum(2.0 * u1 + 2.0 * u2 + 0.1 * u3, 0.0)

    def bn(x):
        cnt = x.shape[0] * x.shape[2]
        s = jnp.sum(jnp.sum(x, axis=2, keepdims=True), axis=0, keepdims=True)
        m = s * (1.0 / cnt)
        dx = x - m
        v = jnp.sum(jnp.sum(dx * dx, axis=2, keepdims=True), axis=0,
                    keepdims=True) * (1.0 / cnt)
        return gamma[None] * dx / jnp.sqrt(v + 1e-5) + beta[None]

    def ib(y, eps_ref):
        mean = y[:, :_OUT, :]
        std = jax.nn.softplus(y[:, _OUT:, :]) + 1e-10
        logstd = jnp.log(std)
        acc = None
        for si in range(_SAMPLE):
            z = mean + std * eps_ref[si]
            e1 = -((z - mean) ** 2) / (2.0 * std * std) - logstd
            diff = jnp.sum(e1 + 0.5 * z * z, axis=1)
            acc = diff if acc is None else acc + diff
        return jnp.sum(acc * (1.0 / _SAMPLE), axis=1, keepdims=True)

    def kl(al):
        term = (al * jnp.log(al / 0.5)
                + (1.0 - al) * jnp.log((1.0 - al) / 0.5))
        return jnp.sum(term, axis=1, keepdims=True)

    w = ixzap_ref.shape[1]
    y_ap = bn(oap_ref[...])
    y_ue = bn(oue_ref[...])
    oap_ref[...] = y_ap
    oue_ref[...] = y_ue
    ixzap_ref[...] = jnp.broadcast_to(ib(y_ap, epsap_ref), (bsz, w))
    ixzue_ref[...] = jnp.broadcast_to(ib(y_ue, epsue_ref), (bsz, w))
    iazap_ref[...] = jnp.broadcast_to(kl(alap_ref[...]), (bsz, w))
    iazue_ref[...] = jnp.broadcast_to(kl(alue_ref[...]), (bsz, w))


def kernel(A_AP, A_UE, H, Graph_AP_reshape, GFA_AP, Graph_UE_reshape, GFA_UE,
           Q1_AP, Q2_AP, Q1_UE, Q2_UE, P1_AP, P1_UE, Att_AP, Att_UE,
           bn_gamma, bn_beta, permutation_size1, permutation_size2, BATCH_SIZE):
    f32 = jnp.float32
    B, D, N1 = A_AP.shape
    N2 = A_UE.shape[2]
    E_ap = Graph_AP_reshape.shape[2]
    E_ue = Graph_UE_reshape.shape[2]
    deg_ap = GFA_AP.shape[2]
    deg_ue = GFA_UE.shape[2]
    O2 = Q1_AP.shape[0]

    kr = jax.random.key(42)
    k1, k2, k3, k4 = jax.random.split(kr, 4)
    u_ap = jax.random.uniform(k1, (B, E_ap), minval=1e-6, maxval=1.0 - 1e-6)
    u_ue = jax.random.uniform(k2, (B, E_ue), minval=1e-6, maxval=1.0 - 1e-6)
    eps_ap = jax.random.normal(k3, (_SAMPLE, B * N1, _OUT))
    eps_ue = jax.random.normal(k4, (_SAMPLE, B * N2, _OUT))
    eps_ap_t = eps_ap.reshape(_SAMPLE, B, N1, _OUT).transpose(0, 1, 3, 2)
    eps_ue_t = eps_ue.reshape(_SAMPLE, B, N2, _OUT).transpose(0, 1, 3, 2)

    gap = Graph_AP_reshape.astype(jnp.int32)
    gue = Graph_UE_reshape.astype(jnp.int32)
    at_ap = jnp.swapaxes(A_AP, 1, 2)
    at_ue = jnp.swapaxes(A_UE, 1, 2)
    gfaT_ap = jnp.swapaxes(GFA_AP, 1, 2).astype(jnp.int32).reshape(B, -1)
    gfaT_ue = jnp.swapaxes(GFA_UE, 1, 2).astype(jnp.int32).reshape(B, -1)
    att_ap = Att_AP.reshape(1, 2 * D).astype(f32)
    att_ue = Att_UE.reshape(1, 2 * D).astype(f32)
    gcol = jnp.broadcast_to(bn_gamma.reshape(O2, 1), (O2, 128)).astype(f32)
    bcol = jnp.broadcast_to(bn_beta.reshape(O2, 1), (O2, 128)).astype(f32)

    p_ap, p_ue, no_ap, no_ue = pl.pallas_call(
        _pre_body,
        out_shape=[
            jax.ShapeDtypeStruct((B, 2, N1), f32),
            jax.ShapeDtypeStruct((B, 2, N2), f32),
            jax.ShapeDtypeStruct((B, E_ap), f32),
            jax.ShapeDtypeStruct((B, E_ue), f32),
        ],
    )(A_AP, A_UE, att_ap, att_ue, u_ap, u_ue)

    npad1 = max(N1, 128)
    npad2 = max(N2, 128)
    p_ap_pad = jnp.concatenate(
        [p_ap, jnp.zeros((B, 2, npad1 - N1), f32)], axis=2) if npad1 > N1 else p_ap
    p_ue_pad = jnp.concatenate(
        [p_ue, jnp.zeros((B, 2, npad2 - N2), f32)], axis=2) if npad2 > N2 else p_ue
    agg_ap, alpha_ap = _make_sc_edge(B, N1, deg_ap, D, E_ap)(
        at_ap, gap, p_ap_pad, no_ap, gfaT_ap)
    agg_ue, alpha_ue = _make_sc_edge(B, N2, deg_ue, D, E_ue)(
        at_ue, gue, p_ue_pad, no_ue, gfaT_ue)

    bspec = lambda shp: pl.BlockSpec(shp, lambda b: (b,) + (0,) * (len(shp) - 1))
    hm1, hm2 = pl.pallas_call(
        _hmean_body,
        grid=(B,),
        in_specs=[bspec((1, _IN2, N1, N2))],
        out_specs=[bspec((1, _IN2, N1)), bspec((1, _IN2, N2))],
        out_shape=[
            jax.ShapeDtypeStruct((B, _IN2, N1), f32),
            jax.ShapeDtypeStruct((B, _IN2, N2), f32),
        ],
        compiler_params=pltpu.CompilerParams(
            dimension_semantics=("arbitrary",)),
    )(H)

    out_ap, out_ue, ixz_ap, ixz_ue, iaz_ap, iaz_ue = pl.pallas_call(
        _final_body,
        out_shape=[
            jax.ShapeDtypeStruct((B, O2, N1), f32),
            jax.ShapeDtypeStruct((B, O2, N2), f32),
            jax.ShapeDtypeStruct((B, 128), f32),
            jax.ShapeDtypeStruct((B, 128), f32),
            jax.ShapeDtypeStruct((B, 128), f32),
            jax.ShapeDtypeStruct((B, 128), f32),
        ],
    )(agg_ap, agg_ue, hm1, hm2, alpha_ap, alpha_ue, eps_ap_t, eps_ue_t,
      Q1_AP, Q2_AP, Q1_UE, Q2_UE, P1_AP, P1_UE, gcol, bcol)

    return (out_ap, out_ue, ixz_ap[:, 0], ixz_ue[:, 0],
            iaz_ap[:, 0], iaz_ue[:, 0])


# speedup vs baseline: 1.3299x; 1.1057x over previous
import numpy as np
import jax
import jax.numpy as jnp
from jax import lax
from jax.experimental import pallas as pl
from jax.experimental.pallas import tpu as pltpu
from jax.experimental.pallas import tpu_sc as plsc

_IN1 = 64
_IN2 = 64
_OUT = 64
_SAMPLE = 2
_LOGIT_CLIP = float(np.log(0.99) - np.log(0.01))
_NC = 2
_NS = 16
_NW = _NC * _NS


def _pre_body(aap_ref, aue_ref, attap_ref, attue_ref, uap_ref, uue_ref,
              pap_ref, pue_ref, nap_ref, nue_ref):
    f32 = jnp.float32
    bsz = aap_ref.shape[0]
    for b in range(bsz):
        for (a_ref, att_ref, p_ref) in ((aap_ref, attap_ref, pap_ref),
                                        (aue_ref, attue_ref, pue_ref)):
            a = a_ref[b]
            p_ref[b, 0:1, :] = jnp.dot(att_ref[:, :_IN1], a,
                                       preferred_element_type=f32)
            p_ref[b, 1:2, :] = jnp.dot(att_ref[:, _IN1:], a,
                                       preferred_element_type=f32)
    for (u_ref, n_ref) in ((uap_ref, nap_ref), (uue_ref, nue_ref)):
        u = u_ref[...]
        n_ref[...] = jnp.log(u) - jnp.log1p(-u)


def _make_sc_edge(bsz, n1, deg1, n2, deg2, d, n_edges):
    tiles_per_b = 16 // bsz
    mesh = plsc.VectorSubcoreMesh(core_axis_name="c", subcore_axis_name="s")
    f32 = jnp.float32
    i32 = jnp.int32
    nmax = max(n1, n2)

    def body(atap_hbm, gap_hbm, pap_hbm, noap_hbm, gfaap_hbm,
             atue_hbm, gue_hbm, pue_hbm, noue_hbm, gfaue_hbm,
             aggap_hbm, alap_hbm, aggue_hbm, alue_hbm,
             at_v, gi_v, gj_v, pi_v, pj_v, no_v, gfa_v, bern_v, al_v, agg_v):
        c = lax.axis_index("c")
        s = lax.axis_index("s")
        b = s // tiles_per_b
        t = s % tiles_per_b

        def side(n, deg, at_hbm, g_hbm, p_hbm, no_hbm, gfa_hbm,
                 agg_hbm, al_hbm):
            npt = n // tiles_per_b
            ept = n_edges // tiles_per_b
            pltpu.sync_copy(at_hbm.at[b], at_v.at[pl.ds(0, n)])
            pltpu.sync_copy(g_hbm.at[b, 0], gj_v)
            pltpu.sync_copy(g_hbm.at[b, 1], gi_v)
            pltpu.sync_copy(p_hbm.at[b, 0], pi_v)
            pltpu.sync_copy(p_hbm.at[b, 1], pj_v)
            pltpu.sync_copy(no_hbm.at[b], no_v)
            pltpu.sync_copy(gfa_hbm.at[b], gfa_v)

            @plsc.parallel_loop(0, n_edges // 16, 1, unroll=4)
            def phase_a(g):
                base = g * 16
                ii = gi_v[pl.ds(base, 16)]
                jj = gj_v[pl.ds(base, 16)]
                s_e = (plsc.load_gather(pi_v, [ii])
                       + plsc.load_gather(pj_v, [jj]))
                lr = jnp.where(s_e >= 0.0, s_e, 0.2 * s_e)
                alpha = jnp.clip(1.0 / (1.0 + jnp.exp(-lr)), 0.01, 0.99)
                logits = jnp.clip(lr, -_LOGIT_CLIP, _LOGIT_CLIP)
                y = (logits + no_v[pl.ds(base, 16)]) * 10.0
                bern_v[pl.ds(base, 16)] = 1.0 / (1.0 + jnp.exp(-y))
                al_v[pl.ds(base, 16)] = alpha

            for ng in range(npt // 16):
                n0 = t * npt + ng * 16
                nl_v = lax.iota(i32, 16) + ng * 16
                for half in range(d // 32):
                    dof = half * 32
                    accs0 = tuple(jnp.zeros((16,), f32) for _ in range(32))

                    @plsc.parallel_loop(0, deg, 1, unroll=2, carry=accs0)
                    def phase_b(k, accs):
                        e_v = gfa_v[pl.ds(k * n + n0, 16)]
                        j_v = plsc.load_gather(gj_v, [e_v])
                        b_v = plsc.load_gather(bern_v, [e_v])
                        out = []
                        for dd in range(32):
                            dv = jnp.full((16,), dd + dof, i32)
                            x = plsc.load_gather(at_v, [j_v, dv])
                            out.append(accs[dd] + x * b_v)
                        return tuple(out)

                    for dd in range(32):
                        dv = jnp.full((16,), dd + dof, i32)
                        plsc.store_scatter(agg_v, [nl_v, dv], phase_b[dd])

            pltpu.sync_copy(agg_v.at[pl.ds(0, npt)],
                            agg_hbm.at[b, pl.ds(t * npt, npt)])
            pltpu.sync_copy(al_v.at[pl.ds(t * ept, ept)],
                            al_hbm.at[b, pl.ds(t * ept, ept)])

        @pl.when(c == 0)
        def _():
            side(n1, deg1, atap_hbm, gap_hbm, pap_hbm, noap_hbm, gfaap_hbm,
                 aggap_hbm, alap_hbm)

        @pl.when(c == 1)
        def _():
            side(n2, deg2, atue_hbm, gue_hbm, pue_hbm, noue_hbm, gfaue_hbm,
                 aggue_hbm, alue_hbm)

    return pl.kernel(
        body,
        out_type=(jax.ShapeDtypeStruct((bsz, n1, d), f32),
                  jax.ShapeDtypeStruct((bsz, n_edges), f32),
                  jax.ShapeDtypeStruct((bsz, n2, d), f32),
                  jax.ShapeDtypeStruct((bsz, n_edges), f32)),
        mesh=mesh,
        compiler_params=pltpu.CompilerParams(needs_layout_passes=False),
        scratch_types=(
            pltpu.VMEM((nmax, d), f32),
            pltpu.VMEM((n_edges,), i32),
            pltpu.VMEM((n_edges,), i32),
            pltpu.VMEM((128,), f32),
            pltpu.VMEM((128,), f32),
            pltpu.VMEM((n_edges,), f32),
            pltpu.VMEM((max(deg1 * n1, deg2 * n2),), i32),
            pltpu.VMEM((n_edges,), f32),
            pltpu.VMEM((n_edges,), f32),
            pltpu.VMEM((nmax // tiles_per_b, d), f32),
        ),
    )


def _hmean_body(h_ref, hm1_ref, hm2_ref):
    h = h_ref[0]
    n1 = h.shape[1]
    n2 = h.shape[2]
    hm1_ref[0] = jnp.sum(h, axis=2) * (1.0 / n2)
    hm2_ref[0] = jnp.sum(h, axis=1) * (1.0 / n1)


def _final_body(aggap_ref, aggue_ref, hm1_ref, hm2_ref, alap_ref, alue_ref,
                epsap_ref, epsue_ref,
                q1ap_ref, q2ap_ref, q1ue_ref, q2ue_ref, p1ap_ref, p1ue_ref,
                g_ref, b_ref,
                oap_ref, oue_ref, ixzap_ref, ixzue_ref, iazap_ref, iazue_ref):
    f32 = jnp.float32
    bsz = aggap_ref.shape[0]
    n1 = aggap_ref.shape[1]
    n2 = aggue_ref.shape[1]
    gamma = g_ref[:, 0:1]
    beta = b_ref[:, 0:1]

    for b in range(bsz):
        agg_ap = aggap_ref[b]
        agg_ue = aggue_ref[b]
        m_ap = jnp.sum(agg_ap, axis=0, keepdims=True) * (1.0 / n1)
        m_ue = jnp.sum(agg_ue, axis=0, keepdims=True) * (1.0 / n2)
        dn = (((1,), (1,)), ((), ()))
        a1 = lax.dot_general(q1ap_ref[...], agg_ap, dn,
                             preferred_element_type=f32)
        a2 = lax.dot_general(q2ap_ref[...], m_ue, dn,
                             preferred_element_type=f32)
        a3 = jnp.dot(p1ap_ref[...], hm1_ref[b], preferred_element_type=f32)
        oap_ref[b] = jnp.maximum(2.0 * a1 + 2.0 * a2 + 0.1 * a3, 0.0)
        u1 = lax.dot_general(q1ue_ref[...], agg_ue, dn,
                             preferred_element_type=f32)
        u2 = lax.dot_general(q2ue_ref[...], m_ap, dn,
                             preferred_element_type=f32)
        u3 = jnp.dot(p1ue_ref[...], hm2_ref[b], preferred_element_type=f32)
        oue_ref[b] = jnp.maximum(2.0 * u1 + 2.0 * u2 + 0.1 * u3, 0.0)

    def bn(x):
        cnt = x.shape[0] * x.shape[2]
        s = jnp.sum(jnp.sum(x, axis=2, keepdims=True), axis=0, keepdims=True)
        m = s * (1.0 / cnt)
        dx = x - m
        v = jnp.sum(jnp.sum(dx * dx, axis=2, keepdims=True), axis=0,
                    keepdims=True) * (1.0 / cnt)
        return gamma[None] * dx / jnp.sqrt(v + 1e-5) + beta[None]

    def ib(y, eps_ref):
        mean = y[:, :_OUT, :]
        std = jax.nn.softplus(y[:, _OUT:, :]) + 1e-10
        logstd = jnp.log(std)
        acc = None
        for si in range(_SAMPLE):
            z = mean + std * eps_ref[si]
            e1 = -((z - mean) ** 2) / (2.0 * std * std) - logstd
            diff = jnp.sum(e1 + 0.5 * z * z, axis=1)
            acc = diff if acc is None else acc + diff
        return jnp.sum(acc * (1.0 / _SAMPLE), axis=1, keepdims=True)

    def kl(al):
        term = (al * jnp.log(al / 0.5)
                + (1.0 - al) * jnp.log((1.0 - al) / 0.5))
        return jnp.sum(term, axis=1, keepdims=True)

    w = ixzap_ref.shape[1]
    y_ap = bn(oap_ref[...])
    y_ue = bn(oue_ref[...])
    oap_ref[...] = y_ap
    oue_ref[...] = y_ue
    ixzap_ref[...] = jnp.broadcast_to(ib(y_ap, epsap_ref), (bsz, w))
    ixzue_ref[...] = jnp.broadcast_to(ib(y_ue, epsue_ref), (bsz, w))
    iazap_ref[...] = jnp.broadcast_to(kl(alap_ref[...]), (bsz, w))
    iazue_ref[...] = jnp.broadcast_to(kl(alue_ref[...]), (bsz, w))


def kernel(A_AP, A_UE, H, Graph_AP_reshape, GFA_AP, Graph_UE_reshape, GFA_UE,
           Q1_AP, Q2_AP, Q1_UE, Q2_UE, P1_AP, P1_UE, Att_AP, Att_UE,
           bn_gamma, bn_beta, permutation_size1, permutation_size2, BATCH_SIZE):
    f32 = jnp.float32
    B, D, N1 = A_AP.shape
    N2 = A_UE.shape[2]
    E_ap = Graph_AP_reshape.shape[2]
    E_ue = Graph_UE_reshape.shape[2]
    deg_ap = GFA_AP.shape[2]
    deg_ue = GFA_UE.shape[2]
    O2 = Q1_AP.shape[0]

    kr = jax.random.key(42)
    k1, k2, k3, k4 = jax.random.split(kr, 4)
    u_ap = jax.random.uniform(k1, (B, E_ap), minval=1e-6, maxval=1.0 - 1e-6)
    u_ue = jax.random.uniform(k2, (B, E_ue), minval=1e-6, maxval=1.0 - 1e-6)
    eps_ap = jax.random.normal(k3, (_SAMPLE, B * N1, _OUT))
    eps_ue = jax.random.normal(k4, (_SAMPLE, B * N2, _OUT))
    eps_ap_t = eps_ap.reshape(_SAMPLE, B, N1, _OUT).transpose(0, 1, 3, 2)
    eps_ue_t = eps_ue.reshape(_SAMPLE, B, N2, _OUT).transpose(0, 1, 3, 2)

    gap = Graph_AP_reshape.astype(jnp.int32)
    gue = Graph_UE_reshape.astype(jnp.int32)
    at_ap = jnp.swapaxes(A_AP, 1, 2)
    at_ue = jnp.swapaxes(A_UE, 1, 2)
    gfaT_ap = jnp.swapaxes(GFA_AP, 1, 2).astype(jnp.int32).reshape(B, -1)
    gfaT_ue = jnp.swapaxes(GFA_UE, 1, 2).astype(jnp.int32).reshape(B, -1)
    att_ap = Att_AP.reshape(1, 2 * D).astype(f32)
    att_ue = Att_UE.reshape(1, 2 * D).astype(f32)
    gcol = jnp.broadcast_to(bn_gamma.reshape(O2, 1), (O2, 128)).astype(f32)
    bcol = jnp.broadcast_to(bn_beta.reshape(O2, 1), (O2, 128)).astype(f32)

    p_ap, p_ue, no_ap, no_ue = pl.pallas_call(
        _pre_body,
        out_shape=[
            jax.ShapeDtypeStruct((B, 2, N1), f32),
            jax.ShapeDtypeStruct((B, 2, N2), f32),
            jax.ShapeDtypeStruct((B, E_ap), f32),
            jax.ShapeDtypeStruct((B, E_ue), f32),
        ],
    )(A_AP, A_UE, att_ap, att_ue, u_ap, u_ue)

    npad1 = max(N1, 128)
    npad2 = max(N2, 128)
    p_ap_pad = jnp.concatenate(
        [p_ap, jnp.zeros((B, 2, npad1 - N1), f32)], axis=2) if npad1 > N1 else p_ap
    p_ue_pad = jnp.concatenate(
        [p_ue, jnp.zeros((B, 2, npad2 - N2), f32)], axis=2) if npad2 > N2 else p_ue
    agg_ap, alpha_ap, agg_ue, alpha_ue = _make_sc_edge(
        B, N1, deg_ap, N2, deg_ue, D, E_ap)(
        at_ap, gap, p_ap_pad, no_ap, gfaT_ap,
        at_ue, gue, p_ue_pad, no_ue, gfaT_ue)

    bspec = lambda shp: pl.BlockSpec(shp, lambda b: (b,) + (0,) * (len(shp) - 1))
    hm1, hm2 = pl.pallas_call(
        _hmean_body,
        grid=(B,),
        in_specs=[bspec((1, _IN2, N1, N2))],
        out_specs=[bspec((1, _IN2, N1)), bspec((1, _IN2, N2))],
        out_shape=[
            jax.ShapeDtypeStruct((B, _IN2, N1), f32),
            jax.ShapeDtypeStruct((B, _IN2, N2), f32),
        ],
        compiler_params=pltpu.CompilerParams(
            dimension_semantics=("arbitrary",)),
    )(H)

    out_ap, out_ue, ixz_ap, ixz_ue, iaz_ap, iaz_ue = pl.pallas_call(
        _final_body,
        out_shape=[
            jax.ShapeDtypeStruct((B, O2, N1), f32),
            jax.ShapeDtypeStruct((B, O2, N2), f32),
            jax.ShapeDtypeStruct((B, 128), f32),
            jax.ShapeDtypeStruct((B, 128), f32),
            jax.ShapeDtypeStruct((B, 128), f32),
            jax.ShapeDtypeStruct((B, 128), f32),
        ],
    )(agg_ap, agg_ue, hm1, hm2, alpha_ap, alpha_ue, eps_ap_t, eps_ue_t,
      Q1_AP, Q2_AP, Q1_UE, Q2_UE, P1_AP, P1_UE, gcol, bcol)

    return (out_ap, out_ue, ixz_ap[:, 0], ixz_ue[:, 0],
            iaz_ap[:, 0], iaz_ue[:, 0])
